# baseline (speedup 1.0000x reference)
import functools

import jax
import jax.numpy as jnp
from jax import lax
from jax.experimental import pallas as pl
from jax.experimental.pallas import tpu as pltpu
from jax.experimental.pallas import tpu_sc as plsc

VOCAB = 100000
EMBED = 64
BATCH = 1024
CTX = 20

NUM_CORES = 2
NUM_SUBCORES = 16
NUM_WORKERS = NUM_CORES * NUM_SUBCORES
B_PER_W = BATCH // NUM_WORKERS
IDX_PER_W = B_PER_W * CTX
IDX_CHUNK = 128
N_CHUNKS = IDX_PER_W // IDX_CHUNK

LANES = 16
EMBED_SLICES = EMBED // LANES


def _pool_body(idx_hbm, e_hbm, pooled_hbm, idx_v, rows_v, pooled_v, sem):
    wid = lax.axis_index("s") * NUM_CORES + lax.axis_index("c")

    pltpu.sync_copy(idx_hbm.at[wid], idx_v)

    descs = [
        pltpu.async_copy(
            e_hbm.at[idx_v.at[j]],
            rows_v.at[pl.ds(j * IDX_CHUNK, IDX_CHUNK)],
            sem,
        )
        for j in range(N_CHUNKS)
    ]
    for d in descs:
        d.wait()

    inv_ctx = jnp.float32(1.0 / CTX)

    def body(i, carry):
        base = i * CTX
        for s in range(EMBED_SLICES):
            acc = rows_v[base, pl.ds(s * LANES, LANES)]
            for c in range(1, CTX):
                acc = acc + rows_v[base + c, pl.ds(s * LANES, LANES)]
            pooled_v[i, pl.ds(s * LANES, LANES)] = acc * inv_ctx
        return carry

    lax.fori_loop(0, B_PER_W, body, 0)

    pltpu.sync_copy(pooled_v, pooled_hbm.at[pl.ds(wid * B_PER_W, B_PER_W)])


def _pool(idx, e):
    pool = pl.kernel(
        _pool_body,
        out_type=jax.ShapeDtypeStruct((BATCH, EMBED), jnp.float32),
        mesh=plsc.VectorSubcoreMesh(core_axis_name="c", subcore_axis_name="s"),
        scratch_types=[
            pltpu.VMEM((N_CHUNKS, IDX_CHUNK), jnp.int32),
            pltpu.VMEM((IDX_PER_W, EMBED), jnp.float32),
            pltpu.VMEM((B_PER_W, EMBED), jnp.float32),
            pltpu.SemaphoreType.DMA,
        ],
        compiler_params=pltpu.CompilerParams(use_tc_tiling_on_sc=False),
    )
    return pool(idx, e)


def _matmul_body(pooled_ref, w_ref, b_ref, out_ref):
    out_ref[...] = (
        jnp.dot(pooled_ref[...], w_ref[...], preferred_element_type=jnp.float32)
        + b_ref[...]
    )


BN = 2048


def _project(pooled, w, b2):
    grid = (pl.cdiv(VOCAB, BN),)
    return pl.pallas_call(
        _matmul_body,
        grid=grid,
        in_specs=[
            pl.BlockSpec((BATCH, EMBED), lambda j: (0, 0)),
            pl.BlockSpec((EMBED, BN), lambda j: (0, j)),
            pl.BlockSpec((1, BN), lambda j: (0, j)),
        ],
        out_specs=pl.BlockSpec((BATCH, BN), lambda j: (0, j)),
        out_shape=jax.ShapeDtypeStruct((BATCH, VOCAB), jnp.float32),
        compiler_params=pltpu.CompilerParams(
            dimension_semantics=("arbitrary",),
        ),
    )(pooled, w, b2)


@jax.jit
def kernel(inputs, E, W, b):
    idx = inputs.astype(jnp.int32).reshape(NUM_WORKERS, N_CHUNKS, IDX_CHUNK)
    pooled = _pool(idx, E)
    return _project(pooled, W, b.reshape(1, VOCAB))


# speedup vs baseline: 1.0099x; 1.0099x over previous
import functools

import jax
import jax.numpy as jnp
from jax import lax
from jax.experimental import pallas as pl
from jax.experimental.pallas import tpu as pltpu
from jax.experimental.pallas import tpu_sc as plsc

VOCAB = 100000
EMBED = 64
BATCH = 1024
CTX = 20

NUM_CORES = 2
NUM_SUBCORES = 16
NUM_WORKERS = NUM_CORES * NUM_SUBCORES
B_PER_W = BATCH // NUM_WORKERS
IDX_PER_W = B_PER_W * CTX
IDX_CHUNK = 128
N_CHUNKS = IDX_PER_W // IDX_CHUNK

LANES = 16
EMBED_SLICES = EMBED // LANES


E_PAD = 128


def _pool_body(idx_hbm, e_hbm, pooled_hbm, idx_v, rows_v, pooled_v, sem):
    wid = lax.axis_index("s") * NUM_CORES + lax.axis_index("c")

    pltpu.sync_copy(idx_hbm.at[wid], idx_v)

    descs = [
        pltpu.async_copy(
            e_hbm.at[idx_v.at[j]],
            rows_v.at[pl.ds(j * IDX_CHUNK, IDX_CHUNK)],
            sem,
        )
        for j in range(N_CHUNKS)
    ]
    for d in descs:
        d.wait()

    inv_ctx = jnp.float32(1.0 / CTX)

    def body(i, carry):
        base = i * CTX
        for s in range(EMBED_SLICES):
            acc = rows_v[base, pl.ds(s * LANES, LANES)]
            for c in range(1, CTX):
                acc = acc + rows_v[base + c, pl.ds(s * LANES, LANES)]
            pooled_v[i, pl.ds(s * LANES, LANES)] = acc * inv_ctx
        return carry

    lax.fori_loop(0, B_PER_W, body, 0)

    pltpu.sync_copy(pooled_v, pooled_hbm.at[pl.ds(wid * B_PER_W, B_PER_W)])


def _pool(idx, e):
    pool = pl.kernel(
        _pool_body,
        out_type=jax.ShapeDtypeStruct((BATCH, EMBED), jnp.float32),
        mesh=plsc.VectorSubcoreMesh(core_axis_name="c", subcore_axis_name="s"),
        scratch_types=[
            pltpu.VMEM((N_CHUNKS, IDX_CHUNK), jnp.int32),
            pltpu.VMEM((IDX_PER_W, E_PAD), jnp.float32),
            pltpu.VMEM((B_PER_W, EMBED), jnp.float32),
            pltpu.SemaphoreType.DMA,
        ],
        compiler_params=pltpu.CompilerParams(use_tc_tiling_on_sc=False),
    )
    return pool(idx, e)


def _matmul_body(pooled_ref, w_ref, b_ref, out_ref):
    out_ref[...] = (
        jnp.dot(pooled_ref[...], w_ref[...], preferred_element_type=jnp.float32)
        + b_ref[...]
    )


BN = 2048


def _project(pooled, w, b2):
    grid = (pl.cdiv(VOCAB, BN),)
    return pl.pallas_call(
        _matmul_body,
        grid=grid,
        in_specs=[
            pl.BlockSpec((BATCH, EMBED), lambda j: (0, 0)),
            pl.BlockSpec((EMBED, BN), lambda j: (0, j)),
            pl.BlockSpec((1, BN), lambda j: (0, j)),
        ],
        out_specs=pl.BlockSpec((BATCH, BN), lambda j: (0, j)),
        out_shape=jax.ShapeDtypeStruct((BATCH, VOCAB), jnp.float32),
        compiler_params=pltpu.CompilerParams(
            dimension_semantics=("arbitrary",),
        ),
    )(pooled, w, b2)


@jax.jit
def kernel(inputs, E, W, b):
    idx = inputs.astype(jnp.int32).reshape(NUM_WORKERS, N_CHUNKS, IDX_CHUNK)
    e_pad = jnp.pad(E, ((0, 0), (0, E_PAD - EMBED)))
    pooled = _pool(idx, e_pad)
    return _project(pooled, W, b.reshape(1, VOCAB))


# speedup vs baseline: 2.6521x; 2.6261x over previous
import functools

import jax
import jax.numpy as jnp
from jax import lax
from jax.experimental import pallas as pl
from jax.experimental.pallas import tpu as pltpu
from jax.experimental.pallas import tpu_sc as plsc

VOCAB = 100000
EMBED = 64
BATCH = 1024
CTX = 20

NUM_CORES = 2
NUM_SUBCORES = 16
NUM_WORKERS = NUM_CORES * NUM_SUBCORES
B_PER_W = BATCH // NUM_WORKERS
IDX_PER_W = B_PER_W * CTX
IDX_CHUNK = 128
N_CHUNKS = IDX_PER_W // IDX_CHUNK

LANES = 16
EMBED_SLICES = EMBED // LANES


E_PAD = 128


def _pool_body(idx_hbm, e_hbm, pooled_hbm, idx_v, rows_v, pooled_v, sem):
    wid = lax.axis_index("s") * NUM_CORES + lax.axis_index("c")

    pltpu.sync_copy(idx_hbm.at[wid], idx_v)

    descs = [
        pltpu.async_copy(
            e_hbm.at[idx_v.at[j]],
            rows_v.at[pl.ds(j * IDX_CHUNK, IDX_CHUNK)],
            sem,
        )
        for j in range(N_CHUNKS)
    ]
    for d in descs:
        d.wait()

    inv_ctx = jnp.float32(1.0 / CTX)

    def body(i, carry):
        base = i * CTX
        for s in range(EMBED_SLICES):
            acc = rows_v[base, pl.ds(s * LANES, LANES)]
            for c in range(1, CTX):
                acc = acc + rows_v[base + c, pl.ds(s * LANES, LANES)]
            pooled_v[i, pl.ds(s * LANES, LANES)] = acc * inv_ctx
        return carry

    lax.fori_loop(0, B_PER_W, body, 0)

    pltpu.sync_copy(pooled_v, pooled_hbm.at[pl.ds(wid * B_PER_W, B_PER_W)])


def _pool(idx, e):
    pool = pl.kernel(
        _pool_body,
        out_type=jax.ShapeDtypeStruct((BATCH, EMBED), jnp.float32),
        mesh=plsc.VectorSubcoreMesh(core_axis_name="c", subcore_axis_name="s"),
        scratch_types=[
            pltpu.VMEM((N_CHUNKS, IDX_CHUNK), jnp.int32),
            pltpu.VMEM((IDX_PER_W, E_PAD), jnp.float32),
            pltpu.VMEM((B_PER_W, EMBED), jnp.float32),
            pltpu.SemaphoreType.DMA,
        ],
        compiler_params=pltpu.CompilerParams(use_tc_tiling_on_sc=False),
    )
    return pool(idx, e)


def _matmul_body(pooled_ref, w_ref, b_ref, out_ref):
    acc = lax.dot_general(
        w_ref[...], pooled_ref[...],
        (((0,), (1,)), ((), ())),
        preferred_element_type=jnp.float32,
    )
    ones = jnp.ones((1, BATCH), jnp.float32)
    bias = lax.dot_general(
        b_ref[...], ones,
        (((0,), (0,)), ((), ())),
        preferred_element_type=jnp.float32,
    )
    out_ref[...] = acc + bias


BN = 2048


def _project_t(pooled, w, b2):
    grid = (pl.cdiv(VOCAB, BN),)
    return pl.pallas_call(
        _matmul_body,
        grid=grid,
        in_specs=[
            pl.BlockSpec((BATCH, EMBED), lambda j: (0, 0)),
            pl.BlockSpec((EMBED, BN), lambda j: (0, j)),
            pl.BlockSpec((1, BN), lambda j: (0, j)),
        ],
        out_specs=pl.BlockSpec((BN, BATCH), lambda j: (j, 0)),
        out_shape=jax.ShapeDtypeStruct((VOCAB, BATCH), jnp.float32),
        compiler_params=pltpu.CompilerParams(
            dimension_semantics=("arbitrary",),
        ),
    )(pooled, w, b2)


@jax.jit
def kernel(inputs, E, W, b):
    idx = inputs.astype(jnp.int32).reshape(NUM_WORKERS, N_CHUNKS, IDX_CHUNK)
    e_pad = jnp.pad(E, ((0, 0), (0, E_PAD - EMBED)))
    pooled = _pool(idx, e_pad)
    return _project_t(pooled, W, b.reshape(1, VOCAB)).T


# speedup vs baseline: 3.3436x; 1.2607x over previous
import jax
import jax.numpy as jnp
from jax import lax
from jax.experimental import pallas as pl
from jax.experimental.pallas import tpu as pltpu
from jax.experimental.pallas import tpu_sc as plsc

VOCAB = 100000
EMBED = 64
BATCH = 1024
CTX = 20

NUM_CORES = 2
NUM_SUBCORES = 16
NUM_WORKERS = NUM_CORES * NUM_SUBCORES
DIMS_PER_W = EMBED // NUM_WORKERS

LANES = 16
N_GROUPS = BATCH // LANES


def _pool_t_body(idxt_hbm, et_hbm, pooledt_hbm, idx_v, row_v, pooled_v, sem):
    wid = lax.axis_index("s") * NUM_CORES + lax.axis_index("c")

    pltpu.sync_copy(idxt_hbm, idx_v)

    inv_ctx = jnp.float32(1.0 / CTX)

    for d_local in range(DIMS_PER_W):
        d = wid * DIMS_PER_W + d_local
        pltpu.async_copy(et_hbm.at[d], row_v, sem).wait()

        def grp(g, carry):
            base = g * LANES
            iv = idx_v[0, pl.ds(base, LANES)]
            acc = plsc.load_gather(row_v, [iv])
            for c in range(1, CTX):
                iv = idx_v[c, pl.ds(base, LANES)]
                acc = acc + plsc.load_gather(row_v, [iv])
            pooled_v[pl.ds(base, LANES)] = acc * inv_ctx
            return carry

        lax.fori_loop(0, N_GROUPS, grp, 0)

        pltpu.sync_copy(pooled_v, pooledt_hbm.at[d])


def _pool_t(idx_t, e_t):
    pool = pl.kernel(
        _pool_t_body,
        out_type=jax.ShapeDtypeStruct((EMBED, BATCH), jnp.float32),
        mesh=plsc.VectorSubcoreMesh(core_axis_name="c", subcore_axis_name="s"),
        scratch_types=[
            pltpu.VMEM((CTX, BATCH), jnp.int32),
            pltpu.VMEM((VOCAB,), jnp.float32),
            pltpu.VMEM((BATCH,), jnp.float32),
            pltpu.SemaphoreType.DMA,
        ],
        compiler_params=pltpu.CompilerParams(needs_layout_passes=False),
    )
    return pool(idx_t, e_t)


def _matmul_body(pooledt_ref, w_ref, b_ref, out_ref):
    acc = lax.dot_general(
        w_ref[...], pooledt_ref[...],
        (((0,), (0,)), ((), ())),
        preferred_element_type=jnp.float32,
    )
    ones = jnp.ones((1, BATCH), jnp.float32)
    bias = lax.dot_general(
        b_ref[...], ones,
        (((0,), (0,)), ((), ())),
        preferred_element_type=jnp.float32,
    )
    out_ref[...] = acc + bias


BN = 2048


def _project_t(pooled_t, w, b2):
    grid = (pl.cdiv(VOCAB, BN),)
    return pl.pallas_call(
        _matmul_body,
        grid=grid,
        in_specs=[
            pl.BlockSpec((EMBED, BATCH), lambda j: (0, 0)),
            pl.BlockSpec((EMBED, BN), lambda j: (0, j)),
            pl.BlockSpec((1, BN), lambda j: (0, j)),
        ],
        out_specs=pl.BlockSpec((BN, BATCH), lambda j: (j, 0)),
        out_shape=jax.ShapeDtypeStruct((VOCAB, BATCH), jnp.float32),
        compiler_params=pltpu.CompilerParams(
            dimension_semantics=("arbitrary",),
        ),
    )(pooled_t, w, b2)


@jax.jit
def kernel(inputs, E, W, b):
    idx_t = inputs.astype(jnp.int32).T
    e_t = E.T
    pooled_t = _pool_t(idx_t, e_t)
    return _project_t(pooled_t, W, b.reshape(1, VOCAB)).T
